# baseline (speedup 1.0000x reference)
import functools

import jax
import jax.numpy as jnp
from jax import lax
from jax.experimental import pallas as pl
from jax.experimental.pallas import tpu as pltpu
from jax.experimental.pallas import tpu_sc as plsc

K = 16
CH = 16
NWORKERS = 32
RPB = 32


def _dist_cand_kernel(hseg_ref, d2_ref, chunk_ref):
    seg = hseg_ref[0]
    s = seg.shape[0]
    nch = s // CH
    gram = jnp.dot(seg, seg.T, preferred_element_type=jnp.float32)
    sq = jnp.sum(seg * seg, axis=1)
    d2 = sq[:, None] + sq[None, :] - 2.0 * gram
    d2_ref[0] = d2
    cm = jnp.max(d2.reshape(nch, CH, s), axis=1)
    bits = lax.bitcast_convert_type(jnp.maximum(cm, 0.0), jnp.int32)
    ci = lax.broadcasted_iota(jnp.int32, (nch, s), 0)
    low = nch - 1
    key = lax.bitcast_convert_type(
        (bits & jnp.int32(~low)) | (low - ci), jnp.float32)
    km = _bitonic_top16(key)
    km = lax.bitcast_convert_type(km, jnp.int32)
    chunk_ref[0] = (low - (km & low)).T


def _cex_pass(x, j, wmf_g):
    r, s = x.shape
    g = r // (2 * j)
    xr = x.reshape(g, 2, j, s)
    a, b = xr[:, 0], xr[:, 1]
    mx = jnp.maximum(a, b)
    mn = jnp.minimum(a, b)
    na = jnp.where(wmf_g, mx, mn)
    nb = jnp.where(wmf_g, mn, mx)
    return jnp.stack((na, nb), axis=1).reshape(r, s)


def _bitonic_top16(x):
    for k in [2, 4, 8, 16]:
        j = k // 2
        while j >= 1:
            g = 64 // (2 * j)
            i0 = lax.broadcasted_iota(jnp.int32, (g, 1, 1), 0) * (2 * j)
            asc_region = ((i0 % 16) & k) == 0
            block_desc = ((i0 >> 4) % 2) == 0
            x = _cex_pass(x, j, asc_region == block_desc)
            j //= 2
    p = jnp.maximum(x[0:16], x[16:32])
    q = jnp.maximum(x[32:48], x[48:64])
    z = jnp.concatenate((p, q), axis=0)
    for j in [8, 4, 2, 1]:
        g = 32 // (2 * j)
        i0 = lax.broadcasted_iota(jnp.int32, (g, 1, 1), 0) * (2 * j)
        z = _cex_pass(z, j, i0 < 16)
    return jnp.maximum(z[0:16], z[16:32])


def _sc_topk(d2rows, chunks, vals_out, idx_out, row_v, ch_v, outv_v, outi_v,
             sem):
    wid = lax.axis_index("s") * 2 + lax.axis_index("c")
    n, s = d2rows.shape
    rows_per_w = n // NWORKERS
    nbatch = rows_per_w // RPB
    base0 = wid * rows_per_w
    pltpu.async_copy(d2rows.at[pl.ds(base0, RPB)], row_v.at[0], sem.at[0])

    def batch_body(bt, _):
        buf = bt % 2
        gbase = base0 + bt * RPB

        @pl.when(bt + 1 < nbatch)
        def _():
            pltpu.async_copy(
                d2rows.at[pl.ds(gbase + RPB, RPB)],
                row_v.at[(bt + 1) % 2], sem.at[(bt + 1) % 2])

        pltpu.make_async_copy(
            d2rows.at[pl.ds(gbase, RPB)], row_v.at[buf], sem.at[buf]).wait()
        pltpu.sync_copy(chunks.at[pl.ds(gbase, RPB)], ch_v)

        def row_body(r, _):
            civ = ch_v[r]
            col0 = civ * CH
            segbase = ((gbase + r) // s) * s
            bfull = jnp.full((K,), buf, jnp.int32)
            rfull = jnp.full((K,), r, jnp.int32)
            pairs = []
            for j in range(CH):
                w = plsc.load_gather(row_v, [bfull, rfull, col0 + j])
                dv = col0 + (segbase + j)
                pairs.append(
                    plsc.sort_key_val(w, dv, descending=(j % 2 == 0)))
            while len(pairs) > 1:
                nxt = []
                for i in range(0, len(pairs), 2):
                    p = i // 2
                    desc = True if len(pairs) == 2 else (p % 2 == 0)
                    av, ai = pairs[i]
                    bv, bi = pairs[i + 1]
                    take = av >= bv
                    mv = jnp.where(take, av, bv)
                    mi = jnp.where(take, ai, bi)
                    nxt.append(plsc.sort_key_val(mv, mi, descending=desc))
                pairs = nxt
            fv, fi = pairs[0]
            outv_v[r] = fv
            outi_v[r] = fi
            return 0

        lax.fori_loop(0, RPB, row_body, 0, unroll=2)
        pltpu.sync_copy(outv_v, vals_out.at[pl.ds(gbase, RPB)])
        pltpu.sync_copy(outi_v, idx_out.at[pl.ds(gbase, RPB)])
        return 0

    lax.fori_loop(0, nbatch, batch_body, 0)


def kernel(h, segs):
    b = segs.shape[0]
    n, d = h.shape
    s = n // b
    hr = h.reshape(b, s, d)
    ngroups = 8
    bg = b // ngroups

    mesh = plsc.VectorSubcoreMesh(core_axis_name="c", subcore_axis_name="s")
    sc = functools.partial(
        pl.kernel,
        mesh=mesh,
        compiler_params=pltpu.CompilerParams(needs_layout_passes=False),
        out_type=[
            jax.ShapeDtypeStruct((bg * s, K), jnp.float32),
            jax.ShapeDtypeStruct((bg * s, K), jnp.int32),
        ],
        scratch_types=[
            pltpu.VMEM((2, RPB, s), jnp.float32),
            pltpu.VMEM((RPB, K), jnp.int32),
            pltpu.VMEM((RPB, K), jnp.float32),
            pltpu.VMEM((RPB, K), jnp.int32),
            pltpu.SemaphoreType.DMA((2,)),
        ],
    )(_sc_topk)

    vparts, iparts = [], []
    for g in range(ngroups):
        d2g, chg = pl.pallas_call(
            _dist_cand_kernel,
            grid=(bg,),
            in_specs=[pl.BlockSpec((1, s, d), lambda i: (i, 0, 0))],
            out_specs=[
                pl.BlockSpec((1, s, s), lambda i: (i, 0, 0)),
                pl.BlockSpec((1, s, K), lambda i: (i, 0, 0)),
            ],
            out_shape=[
                jax.ShapeDtypeStruct((bg, s, s), jnp.float32),
                jax.ShapeDtypeStruct((bg, s, K), jnp.int32),
            ],
        )(hr[g * bg:(g + 1) * bg])
        vg, ig = sc(d2g.reshape(bg * s, s), chg.reshape(bg * s, K))
        vparts.append(vg)
        iparts.append(ig + g * bg * s)

    vals = jnp.concatenate(vparts, axis=0).reshape(b, s, K)
    dst = jnp.concatenate(iparts, axis=0).reshape(-1)
    src = jnp.broadcast_to(
        jnp.arange(n, dtype=jnp.int32).reshape(b, s, 1), (b, s, K)
    ).reshape(-1)
    return src, dst, vals, h


# speedup vs baseline: 1.2091x; 1.2091x over previous
import functools

import jax
import jax.numpy as jnp
from jax import lax
from jax.experimental import pallas as pl
from jax.experimental.pallas import tpu as pltpu
from jax.experimental.pallas import tpu_sc as plsc

K = 16
CH = 16
NWORKERS = 32
RPB = 32


def _dist_cand_kernel(hseg_ref, d2_ref, chunk_ref):
    seg = hseg_ref[0]
    s = seg.shape[0]
    nch = s // CH
    gram = jnp.dot(seg, seg.T, preferred_element_type=jnp.float32)
    sq = jnp.sum(seg * seg, axis=1)
    d2 = sq[:, None] + sq[None, :] - 2.0 * gram
    d2_ref[0] = d2
    cm = jnp.max(d2.reshape(nch, CH, s), axis=1)
    bits = lax.bitcast_convert_type(jnp.maximum(cm, 0.0), jnp.int32)
    ci = lax.broadcasted_iota(jnp.int32, (nch, s), 0)
    low = nch - 1
    key = lax.bitcast_convert_type(
        (bits & jnp.int32(~low)) | (low - ci), jnp.float32)
    km = _bitonic_top16(key)
    km = lax.bitcast_convert_type(km, jnp.int32)
    chunk_ref[0] = (low - (km & low)).T


def _cex_pass(x, j, wmf_g):
    r, s = x.shape
    g = r // (2 * j)
    xr = x.reshape(g, 2, j, s)
    a, b = xr[:, 0], xr[:, 1]
    mx = jnp.maximum(a, b)
    mn = jnp.minimum(a, b)
    na = jnp.where(wmf_g, mx, mn)
    nb = jnp.where(wmf_g, mn, mx)
    return jnp.stack((na, nb), axis=1).reshape(r, s)


def _bitonic_top16(x):
    for k in [2, 4, 8, 16]:
        j = k // 2
        while j >= 1:
            g = 64 // (2 * j)
            i0 = lax.broadcasted_iota(jnp.int32, (g, 1, 1), 0) * (2 * j)
            asc_region = ((i0 % 16) & k) == 0
            block_desc = ((i0 >> 4) % 2) == 0
            x = _cex_pass(x, j, asc_region == block_desc)
            j //= 2
    p = jnp.maximum(x[0:16], x[16:32])
    q = jnp.maximum(x[32:48], x[48:64])
    z = jnp.concatenate((p, q), axis=0)
    for j in [8, 4, 2, 1]:
        g = 32 // (2 * j)
        i0 = lax.broadcasted_iota(jnp.int32, (g, 1, 1), 0) * (2 * j)
        z = _cex_pass(z, j, i0 < 16)
    return jnp.maximum(z[0:16], z[16:32])


def _sc_topk(d2rows, chunks, vals_out, idx_out, row_v, ch_v, outv_v, outi_v,
             sem):
    wid = lax.axis_index("s") * 2 + lax.axis_index("c")
    n, s = d2rows.shape
    rows_per_w = n // NWORKERS
    nbatch = rows_per_w // RPB
    base0 = wid * rows_per_w
    pltpu.async_copy(d2rows.at[pl.ds(base0, RPB)], row_v.at[0], sem.at[0])

    def batch_body(bt, _):
        buf = bt % 2
        gbase = base0 + bt * RPB

        @pl.when(bt + 1 < nbatch)
        def _():
            pltpu.async_copy(
                d2rows.at[pl.ds(gbase + RPB, RPB)],
                row_v.at[(bt + 1) % 2], sem.at[(bt + 1) % 2])

        pltpu.make_async_copy(
            d2rows.at[pl.ds(gbase, RPB)], row_v.at[buf], sem.at[buf]).wait()
        pltpu.sync_copy(chunks.at[pl.ds(gbase, RPB)], ch_v)

        def row_body(r, _):
            civ = ch_v[r]
            col0 = civ * CH
            segbase = ((gbase + r) // s) * s
            bfull = jnp.full((K,), buf, jnp.int32)
            rfull = jnp.full((K,), r, jnp.int32)
            pairs = []
            for j in range(CH):
                w = plsc.load_gather(row_v, [bfull, rfull, col0 + j])
                dv = col0 + (segbase + j)
                pairs.append(
                    plsc.sort_key_val(w, dv, descending=(j % 2 == 0)))
            while len(pairs) > 1:
                nxt = []
                for i in range(0, len(pairs), 2):
                    p = i // 2
                    desc = True if len(pairs) == 2 else (p % 2 == 0)
                    av, ai = pairs[i]
                    bv, bi = pairs[i + 1]
                    take = av >= bv
                    mv = jnp.where(take, av, bv)
                    mi = jnp.where(take, ai, bi)
                    nxt.append(plsc.sort_key_val(mv, mi, descending=desc))
                pairs = nxt
            fv, fi = pairs[0]
            outv_v[r] = fv
            outi_v[r] = fi
            return 0

        lax.fori_loop(0, RPB, row_body, 0, unroll=2)
        pltpu.sync_copy(outv_v, vals_out.at[pl.ds(gbase, RPB)])
        pltpu.sync_copy(outi_v, idx_out.at[pl.ds(gbase, RPB)])
        return 0

    lax.fori_loop(0, nbatch, batch_body, 0)


def kernel(h, segs):
    b = segs.shape[0]
    n, d = h.shape
    s = n // b
    hr = h.reshape(b, s, d)
    ngroups = 2
    bg = b // ngroups

    mesh = plsc.VectorSubcoreMesh(core_axis_name="c", subcore_axis_name="s")
    sc = functools.partial(
        pl.kernel,
        mesh=mesh,
        compiler_params=pltpu.CompilerParams(needs_layout_passes=False),
        out_type=[
            jax.ShapeDtypeStruct((bg * s, K), jnp.float32),
            jax.ShapeDtypeStruct((bg * s, K), jnp.int32),
        ],
        scratch_types=[
            pltpu.VMEM((2, RPB, s), jnp.float32),
            pltpu.VMEM((RPB, K), jnp.int32),
            pltpu.VMEM((RPB, K), jnp.float32),
            pltpu.VMEM((RPB, K), jnp.int32),
            pltpu.SemaphoreType.DMA((2,)),
        ],
    )(_sc_topk)

    vparts, iparts = [], []
    for g in range(ngroups):
        d2g, chg = pl.pallas_call(
            _dist_cand_kernel,
            grid=(bg,),
            in_specs=[pl.BlockSpec((1, s, d), lambda i: (i, 0, 0))],
            out_specs=[
                pl.BlockSpec((1, s, s), lambda i: (i, 0, 0)),
                pl.BlockSpec((1, s, K), lambda i: (i, 0, 0)),
            ],
            out_shape=[
                jax.ShapeDtypeStruct((bg, s, s), jnp.float32),
                jax.ShapeDtypeStruct((bg, s, K), jnp.int32),
            ],
        )(hr[g * bg:(g + 1) * bg])
        vg, ig = sc(d2g.reshape(bg * s, s), chg.reshape(bg * s, K))
        vparts.append(vg)
        iparts.append(ig + g * bg * s)

    vals = jnp.concatenate(vparts, axis=0).reshape(b, s, K)
    dst = jnp.concatenate(iparts, axis=0).reshape(-1)
    src = jnp.broadcast_to(
        jnp.arange(n, dtype=jnp.int32).reshape(b, s, 1), (b, s, K)
    ).reshape(-1)
    return src, dst, vals, h


# speedup vs baseline: 1.2506x; 1.0343x over previous
import functools

import jax
import jax.numpy as jnp
from jax import lax
from jax.experimental import pallas as pl
from jax.experimental.pallas import tpu as pltpu
from jax.experimental.pallas import tpu_sc as plsc

K = 16
CH = 16
NWORKERS = 32
RPB = 32


def _dist_cand_kernel(hseg_ref, d2_ref, chunk_ref):
    seg = hseg_ref[0]
    s = seg.shape[0]
    nch = s // CH
    gram = jnp.dot(seg, seg.T, preferred_element_type=jnp.float32)
    sq = jnp.sum(seg * seg, axis=1)
    d2 = sq[:, None] + sq[None, :] - 2.0 * gram
    d2_ref[0] = d2
    cm = jnp.max(d2.reshape(nch, CH, s), axis=1)
    bits = lax.bitcast_convert_type(jnp.maximum(cm, 0.0), jnp.int32)
    ci = lax.broadcasted_iota(jnp.int32, (nch, s), 0)
    low = nch - 1
    key = lax.bitcast_convert_type(
        (bits & jnp.int32(~low)) | (low - ci), jnp.float32)
    km = _bitonic_top16(key)
    km = lax.bitcast_convert_type(km, jnp.int32)
    chunk_ref[0] = (low - (km & low)).T


def _cex_pass(x, j, wmf_g):
    r, s = x.shape
    g = r // (2 * j)
    xr = x.reshape(g, 2, j, s)
    a, b = xr[:, 0], xr[:, 1]
    mx = jnp.maximum(a, b)
    mn = jnp.minimum(a, b)
    na = jnp.where(wmf_g, mx, mn)
    nb = jnp.where(wmf_g, mn, mx)
    return jnp.stack((na, nb), axis=1).reshape(r, s)


def _bitonic_top16(x):
    for k in [2, 4, 8, 16]:
        j = k // 2
        while j >= 1:
            g = 64 // (2 * j)
            i0 = lax.broadcasted_iota(jnp.int32, (g, 1, 1), 0) * (2 * j)
            asc_region = ((i0 % 16) & k) == 0
            block_desc = ((i0 >> 4) % 2) == 0
            x = _cex_pass(x, j, asc_region == block_desc)
            j //= 2
    p = jnp.maximum(x[0:16], x[16:32])
    q = jnp.maximum(x[32:48], x[48:64])
    z = jnp.concatenate((p, q), axis=0)
    for j in [8, 4, 2, 1]:
        g = 32 // (2 * j)
        i0 = lax.broadcasted_iota(jnp.int32, (g, 1, 1), 0) * (2 * j)
        z = _cex_pass(z, j, i0 < 16)
    return jnp.maximum(z[0:16], z[16:32])


def _sc_topk(d2rows, chunks, vals_out, idx_out, row_v, ch_v, outv_v, outi_v,
             sem):
    wid = lax.axis_index("s") * 2 + lax.axis_index("c")
    n, s = d2rows.shape
    rows_per_w = n // NWORKERS
    nbatch = rows_per_w // RPB
    base0 = wid * rows_per_w
    pltpu.async_copy(d2rows.at[pl.ds(base0, RPB)], row_v.at[0], sem.at[0])

    def batch_body(bt, _):
        buf = bt % 2
        gbase = base0 + bt * RPB

        @pl.when(bt + 1 < nbatch)
        def _():
            pltpu.async_copy(
                d2rows.at[pl.ds(gbase + RPB, RPB)],
                row_v.at[(bt + 1) % 2], sem.at[(bt + 1) % 2])

        pltpu.make_async_copy(
            d2rows.at[pl.ds(gbase, RPB)], row_v.at[buf], sem.at[buf]).wait()
        pltpu.sync_copy(chunks.at[pl.ds(gbase, RPB)], ch_v)

        def row_body(r, _):
            civ = ch_v[r]
            col0 = civ * CH
            segbase = ((gbase + r) // s) * s
            bfull = jnp.full((K,), buf, jnp.int32)
            rfull = jnp.full((K,), r, jnp.int32)
            pairs = []
            for j in range(CH):
                w = plsc.load_gather(row_v, [bfull, rfull, col0 + j])
                dv = col0 + (segbase + j)
                pairs.append(
                    plsc.sort_key_val(w, dv, descending=(j % 2 == 0)))
            while len(pairs) > 1:
                nxt = []
                for i in range(0, len(pairs), 2):
                    p = i // 2
                    desc = True if len(pairs) == 2 else (p % 2 == 0)
                    av, ai = pairs[i]
                    bv, bi = pairs[i + 1]
                    take = av >= bv
                    mv = jnp.where(take, av, bv)
                    mi = jnp.where(take, ai, bi)
                    nxt.append(plsc.sort_key_val(mv, mi, descending=desc))
                pairs = nxt
            fv, fi = pairs[0]
            outv_v[r] = fv
            outi_v[r] = fi
            return 0

        lax.fori_loop(0, RPB, row_body, 0, unroll=2)
        pltpu.sync_copy(outv_v, vals_out.at[pl.ds(gbase, RPB)])
        pltpu.sync_copy(outi_v, idx_out.at[pl.ds(gbase, RPB)])
        return 0

    lax.fori_loop(0, nbatch, batch_body, 0)


def kernel(h, segs):
    b = segs.shape[0]
    n, d = h.shape
    s = n // b
    hr = h.reshape(b, s, d)
    ngroups = 1
    bg = b // ngroups

    mesh = plsc.VectorSubcoreMesh(core_axis_name="c", subcore_axis_name="s")
    sc = functools.partial(
        pl.kernel,
        mesh=mesh,
        compiler_params=pltpu.CompilerParams(needs_layout_passes=False),
        out_type=[
            jax.ShapeDtypeStruct((bg * s, K), jnp.float32),
            jax.ShapeDtypeStruct((bg * s, K), jnp.int32),
        ],
        scratch_types=[
            pltpu.VMEM((2, RPB, s), jnp.float32),
            pltpu.VMEM((RPB, K), jnp.int32),
            pltpu.VMEM((RPB, K), jnp.float32),
            pltpu.VMEM((RPB, K), jnp.int32),
            pltpu.SemaphoreType.DMA((2,)),
        ],
    )(_sc_topk)

    vparts, iparts = [], []
    for g in range(ngroups):
        d2g, chg = pl.pallas_call(
            _dist_cand_kernel,
            grid=(bg,),
            in_specs=[pl.BlockSpec((1, s, d), lambda i: (i, 0, 0))],
            out_specs=[
                pl.BlockSpec((1, s, s), lambda i: (i, 0, 0)),
                pl.BlockSpec((1, s, K), lambda i: (i, 0, 0)),
            ],
            out_shape=[
                jax.ShapeDtypeStruct((bg, s, s), jnp.float32),
                jax.ShapeDtypeStruct((bg, s, K), jnp.int32),
            ],
        )(hr[g * bg:(g + 1) * bg])
        vg, ig = sc(d2g.reshape(bg * s, s), chg.reshape(bg * s, K))
        vparts.append(vg)
        iparts.append(ig + g * bg * s)

    vals = jnp.concatenate(vparts, axis=0).reshape(b, s, K)
    dst = jnp.concatenate(iparts, axis=0).reshape(-1)
    src = jnp.broadcast_to(
        jnp.arange(n, dtype=jnp.int32).reshape(b, s, 1), (b, s, K)
    ).reshape(-1)
    return src, dst, vals, h
